# Initial kernel scaffold; baseline (speedup 1.0000x reference)
import jax
import jax.numpy as jnp
from jax.experimental import pallas as pl


def kernel(pillar_features, voxel_coords, observations, conv_w, bn_gamma, bn_beta):
    raise NotImplementedError("write your pallas kernel here")


# speedup vs baseline: 3.4826x; 3.4826x over previous
import functools

import jax
import jax.numpy as jnp
from jax import lax
from jax.experimental import pallas as pl
from jax.experimental.pallas import tpu as pltpu
from jax.experimental.pallas import tpu_sc as plsc

B = 4
NX = 432
NY = 496
C = 64
P = 40000
PLANE = NY * NX
NW = 32
SLAB = (B * PLANE) // NW
ROWS = SLAB // NX
TPB = NW // B
L = 16
CH = 4000
CAP = 448


def _sc_scatter_body(vc_hbm, pf_hbm, out_hbm, vcl, W, Lp, Lx, T, G, sem):
    wid = lax.axis_index("c") * 16 + lax.axis_index("s")
    base = wid * SLAB
    b_idx = wid // TPB
    y0 = (wid % TPB) * ROWS
    lanes = lax.iota(jnp.int32, L)
    zero16f = jnp.zeros((L,), jnp.float32)
    zero16i = jnp.zeros((L,), jnp.int32)

    def _initw(i, c):
        W[pl.ds(i * L, L)] = jnp.full((L,), -1, jnp.int32)
        return c
    lax.fori_loop(0, SLAB // L, _initw, 0)

    def _initt(q, c):
        T[q // (NX // L), pl.ds((q % (NX // L)) * L, L)] = zero16f
        return c
    lax.fori_loop(0, C * (NX // L), _initt, 0)

    def _chunk(q, c):
        pltpu.sync_copy(vc_hbm.at[pl.ds(q * CH * 4, CH * 4)], vcl)

        def _vec(t, c2):
            row4 = (t * L + lanes) * 4
            bb = plsc.load_gather(vcl, [row4])
            yy = plsc.load_gather(vcl, [row4 + 2])
            xx = plsc.load_gather(vcl, [row4 + 3])
            off = bb * PLANE + yy * NX + xx - base
            msk = (off >= 0) & (off < SLAB)
            pid = q * CH + t * L + lanes
            plsc.store_scatter(W, [off], pid, mask=msk)
            return c2
        lax.fori_loop(0, CH // L, _vec, 0)
        return c
    lax.fori_loop(0, P // CH, _chunk, 0)

    def _row(r, c):
        def _zl(j, c2):
            Lp[pl.ds(j * L, L)] = zero16i
            return c2
        lax.fori_loop(0, CAP // L, _zl, 0)

        def _scan(k, cnt):
            wv = W[pl.ds(r * NX + k * L, L)]
            msk = wv >= 0
            xv = k * L + lanes
            plsc.store_compressed(Lp.at[pl.ds(cnt, L)], wv, mask=msk)
            plsc.store_compressed(Lx.at[pl.ds(cnt, L)], xv, mask=msk)
            pc = plsc.all_reduce_population_count(msk)
            return cnt + (pc[0] if pc.ndim else pc)
        m = lax.fori_loop(0, NX // L, _scan, 0)

        def _wave(w, c2):
            idxv = Lp[pl.ds(w * L, L)]
            pltpu.async_copy(pf_hbm.at[idxv], G, sem).wait()
            xv = Lx[pl.ds(w * L, L)]
            for i in range(L):
                @pl.when(w * L + i < m)
                def _():
                    xs = jnp.full((L,), xv[i], jnp.int32)
                    for j in range(C // L):
                        plsc.store_scatter(
                            T, [j * L + lanes, xs], G[i, pl.ds(j * L, L)])
            return c2
        nw = (m + L - 1) // L
        lax.fori_loop(0, nw, _wave, 0)

        pltpu.sync_copy(
            T, out_hbm.at[b_idx, pl.ds(0, C), pl.ds((y0 + r) * NX, NX)])

        def _undo(w, c2):
            xv = Lx[pl.ds(w * L, L)]
            for i in range(L):
                @pl.when(w * L + i < m)
                def _():
                    xs = jnp.full((L,), xv[i], jnp.int32)
                    for j in range(C // L):
                        plsc.store_scatter(T, [j * L + lanes, xs], zero16f)
            return c2
        lax.fori_loop(0, nw, _undo, 0)
        return c
    lax.fori_loop(0, ROWS, _row, 0)


@functools.partial(jax.jit, static_argnums=())
def _sc_scatter(vc, pf):
    mesh = plsc.VectorSubcoreMesh(core_axis_name="c", subcore_axis_name="s",
                                  num_cores=2, num_subcores=16)
    f = pl.kernel(
        _sc_scatter_body,
        out_type=jax.ShapeDtypeStruct((B, 2 * C, PLANE), jnp.float32),
        mesh=mesh,
        scratch_types=[
            pltpu.VMEM((CH * 4,), jnp.int32),
            pltpu.VMEM((SLAB,), jnp.int32),
            pltpu.VMEM((CAP,), jnp.int32),
            pltpu.VMEM((CAP,), jnp.int32),
            pltpu.VMEM((C, NX), jnp.float32),
            pltpu.VMEM((L, C), jnp.float32),
            pltpu.SemaphoreType.DMA,
        ],
        compiler_params=pltpu.CompilerParams(use_tc_tiling_on_sc=False,
                                             needs_layout_passes=False),
    )
    return f(vc, pf)


BLK = 16
W_BLK = BLK * NX
HALO = 512
SEG = PLANE + 2 * HALO


def _windows(pad_ref, b, rb, xpos):
    v = pad_ref[0:1, pl.ds(b * SEG + rb * W_BLK, W_BLK + 2 * HALO)]
    rows = []
    for k in range(9):
        dy, dx = k // 3, k % 3
        delta = (dy - 1) * NX + (dx - 1)
        win = jnp.roll(v, -(HALO + delta), axis=1)[:, :W_BLK]
        if dx == 0:
            win = jnp.where(xpos == 0, 0.0, win)
        elif dx == 2:
            win = jnp.where(xpos == NX - 1, 0.0, win)
        rows.append(win)
    return rows


def _fill_pad(pad_ref, obs_ref, b, rb):
    @pl.when((b == 0) & (rb == 0))
    def _():
        for bb in range(B):
            pad_ref[0:1, pl.ds(bb * SEG, HALO)] = jnp.zeros(
                (1, HALO), jnp.float32)
            pad_ref[0:1, pl.ds(bb * SEG + HALO, PLANE)] = obs_ref[bb:bb + 1, :]
            pad_ref[0:1, pl.ds(bb * SEG + HALO + PLANE, HALO)] = jnp.zeros(
                (1, HALO), jnp.float32)


def _k1_body(obs_ref, out_ref, pad_ref):
    b = pl.program_id(0)
    rb = pl.program_id(1)
    _fill_pad(pad_ref, obs_ref, b, rb)
    xpos = lax.broadcasted_iota(jnp.int32, (1, W_BLK), 1) % NX
    rows = _windows(pad_ref, b, rb, xpos)
    rows.append(jnp.ones((1, W_BLK), jnp.float32))
    rows.append(jnp.zeros((6, W_BLK), jnp.float32))
    S = jnp.concatenate(rows, axis=0)
    contrib = lax.dot_general(S, S, (((1,), (1,)), ((), ())),
                              preferred_element_type=jnp.float32)

    @pl.when((b == 0) & (rb == 0))
    def _():
        out_ref[...] = jnp.zeros((L, L), jnp.float32)
    out_ref[...] += contrib


def _k1_moments(obs_flat):
    return pl.pallas_call(
        _k1_body,
        grid=(B, NY // BLK),
        in_specs=[pl.BlockSpec((B, PLANE), lambda b, r: (0, 0))],
        out_specs=pl.BlockSpec((L, L), lambda b, r: (0, 0)),
        out_shape=jax.ShapeDtypeStruct((L, L), jnp.float32),
        scratch_shapes=[pltpu.VMEM((1, B * SEG), jnp.float32)],
    )(obs_flat)


def _k2_body(bev_ref, obs_ref, m16_ref, w9_ref, g_ref, bt_ref, out_ref,
             pad_ref):
    del bev_ref
    b = pl.program_id(0)
    rb = pl.program_id(1)
    _fill_pad(pad_ref, obs_ref, b, rb)

    mv = m16_ref[...]
    w9 = w9_ref[...]
    cnt = mv[9:10, 9:10]
    mean = lax.dot_general(w9, mv[9:10, 0:9],
                           (((1,), (1,)), ((), ())),
                           preferred_element_type=jnp.float32) / cnt
    t1 = lax.dot_general(w9, mv[0:9, 0:9], (((1,), (0,)), ((), ())),
                         preferred_element_type=jnp.float32)
    e2 = jnp.sum(t1 * w9, axis=1, keepdims=True) / cnt
    var = e2 - mean * mean
    alpha = g_ref[...] * lax.rsqrt(var + 1e-3)
    w_eff = w9 * alpha
    b_eff = bt_ref[...] - alpha * mean

    xpos = lax.broadcasted_iota(jnp.int32, (1, W_BLK), 1) % NX
    S = jnp.concatenate(_windows(pad_ref, b, rb, xpos), axis=0)
    y = lax.dot_general(w_eff, S, (((1,), (0,)), ((), ())),
                        preferred_element_type=jnp.float32)
    out_ref[...] = jnp.maximum(y + b_eff, 0.0)[None]


def _k2_conv(bev_buf, obs_flat, m16, w9, g64, b64):
    return pl.pallas_call(
        _k2_body,
        grid=(B, NY // BLK),
        in_specs=[
            pl.BlockSpec(memory_space=pl.ANY),
            pl.BlockSpec((B, PLANE), lambda b, r: (0, 0)),
            pl.BlockSpec((L, L), lambda b, r: (0, 0)),
            pl.BlockSpec((C, 9), lambda b, r: (0, 0)),
            pl.BlockSpec((C, 1), lambda b, r: (0, 0)),
            pl.BlockSpec((C, 1), lambda b, r: (0, 0)),
        ],
        out_specs=pl.BlockSpec((1, C, W_BLK), lambda b, r: (b, 1, r)),
        out_shape=jax.ShapeDtypeStruct((B, 2 * C, PLANE), jnp.float32),
        input_output_aliases={0: 0},
        scratch_shapes=[pltpu.VMEM((1, B * SEG), jnp.float32)],
    )(bev_buf, obs_flat, m16, w9, g64, b64)


def kernel(pillar_features, voxel_coords, observations, conv_w, bn_gamma,
           bn_beta):
    vc = voxel_coords.astype(jnp.int32).reshape(P * 4)
    obs_flat = observations.reshape(B, PLANE)
    w9 = conv_w.reshape(C, 9).astype(jnp.float32)
    g64 = bn_gamma.reshape(C, 1).astype(jnp.float32)
    b64 = bn_beta.reshape(C, 1).astype(jnp.float32)
    bev_buf = _sc_scatter(vc, pillar_features)
    m16 = _k1_moments(obs_flat)
    out = _k2_conv(bev_buf, obs_flat, m16, w9, g64, b64)
    return out.reshape(B, 2 * C, NY, NX)


# speedup vs baseline: 4.5529x; 1.3073x over previous
import functools

import jax
import jax.numpy as jnp
from jax import lax
from jax.experimental import pallas as pl
from jax.experimental.pallas import tpu as pltpu
from jax.experimental.pallas import tpu_sc as plsc

B = 4
NX = 432
NY = 496
C = 64
P = 40000
PLANE = NY * NX
L = 16
CH = 2000
NT = 8
SLAB_ROWS = NT * 8
SLAB = SLAB_ROWS * NX
TSZ = 8 * NX
CAPT = TSZ + L
GC = 160


def _sc_scatter_body(vc_hbm, pf2_hbm, out_hbm, vcl, W, Lp, Lpos, T, Gbig,
                     sem):
    wid = lax.axis_index("c") * 16 + lax.axis_index("s")
    b_idx = wid // 8
    yt0 = jnp.minimum((wid % 8) * NT, 62 - NT)
    y_start = yt0 * 8
    lanes = lax.iota(jnp.int32, L)
    zero16f = jnp.zeros((L,), jnp.float32)
    zero16i = jnp.zeros((L,), jnp.int32)

    def _initw(i, c):
        W[pl.ds(i * L, L)] = jnp.full((L,), -1, jnp.int32)
        return c
    lax.fori_loop(0, SLAB // L, _initw, 0)

    def _initt(q, c):
        cc = q // (TSZ // L)
        rem = q % (TSZ // L)
        T[cc, rem // (NX // L), pl.ds((rem % (NX // L)) * L, L)] = zero16f
        return c
    lax.fori_loop(0, L * (TSZ // L), _initt, 0)

    def _chunk(q, c):
        pltpu.sync_copy(vc_hbm.at[pl.ds(q * CH * 4, CH * 4)], vcl)

        def _vec(t, c2):
            row4 = (t * L + lanes) * 4
            bb = plsc.load_gather(vcl, [row4])
            yy = plsc.load_gather(vcl, [row4 + 2])
            xx = plsc.load_gather(vcl, [row4 + 3])
            off = (yy - y_start) * NX + xx
            msk = (bb == b_idx) & (yy >= y_start) & (yy < y_start + SLAB_ROWS)
            pid = q * CH + t * L + lanes
            plsc.store_scatter(W, [off], pid, mask=msk)
            return c2
        lax.fori_loop(0, CH // L, _vec, 0)
        return c
    lax.fori_loop(0, P // CH, _chunk, 0)

    def _tile(t, c):
        def _zl(j, c2):
            Lp[pl.ds(j * L, L)] = zero16i
            return c2
        lax.fori_loop(0, CAPT // L, _zl, 0)

        def _scan(k, cnt):
            wv = W[pl.ds(t * TSZ + k * L, L)]
            msk = wv >= 0
            posv = k * L + lanes
            plsc.store_compressed(Lp.at[pl.ds(cnt, L)], wv, mask=msk)
            plsc.store_compressed(Lpos.at[pl.ds(cnt, L)], posv, mask=msk)
            pc = plsc.all_reduce_population_count(msk)
            return cnt + (pc[0] if pc.ndim else pc)
        m = lax.fori_loop(0, TSZ // L, _scan, 0)
        n_super = (m + GC - 1) // GC

        def _wave_entries(s, w, fn):
            lp_v = Lp[pl.ds(s * GC + w * L, L)]
            pos_v = Lpos[pl.ds(s * GC + w * L, L)]
            for i in range(L):
                @pl.when(s * GC + w * L + i < m)
                def _():
                    p = lp_v[i]
                    pos = pos_v[i]
                    yloc = pos // NX
                    xx = pos - yloc * NX
                    fn(p, w * L + i, jnp.full((L,), yloc, jnp.int32),
                       jnp.full((L,), xx, jnp.int32))

        for g in range(4):
            def _super(s, c2):
                @pl.when((g == 0) | (n_super > 1))
                def _():
                    k_s = jnp.minimum(m - s * GC, GC)
                    nw = (k_s + L - 1) // L

                    def _fire(w, c3):
                        rows = Lp[pl.ds(s * GC + w * L, L)] // 2
                        pltpu.async_copy(
                            pf2_hbm.at[rows], Gbig.at[pl.ds(w * L, L)],
                            sem).wait()
                        return c3
                    lax.fori_loop(0, nw, _fire, 0)

                def _scat(p, slot, ysp, xsp):
                    vals = Gbig[slot, pl.ds((p % 2) * C + g * L, L)]
                    plsc.store_scatter(T, [lanes, ysp, xsp], vals)

                def _swave(w, c3):
                    _wave_entries(s, w, _scat)
                    return c3
                k_s2 = jnp.minimum(m - s * GC, GC)
                lax.fori_loop(0, (k_s2 + L - 1) // L, _swave, 0)
                return c2
            lax.fori_loop(0, n_super, _super, 0)

            pltpu.sync_copy(
                T, out_hbm.at[b_idx, pl.ds(g * L, L),
                              pl.ds((yt0 + t) * 8, 8), :])

            def _undo_s(s, c2):
                def _uscat(p, slot, ysp, xsp):
                    plsc.store_scatter(T, [lanes, ysp, xsp], zero16f)

                def _uwave(w, c3):
                    _wave_entries(s, w, _uscat)
                    return c3
                k_s2 = jnp.minimum(m - s * GC, GC)
                lax.fori_loop(0, (k_s2 + L - 1) // L, _uwave, 0)
                return c2
            lax.fori_loop(0, n_super, _undo_s, 0)
        return c
    lax.fori_loop(0, NT, _tile, 0)


def _sc_scatter(vc, pf2):
    mesh = plsc.VectorSubcoreMesh(core_axis_name="c", subcore_axis_name="s",
                                  num_cores=2, num_subcores=16)
    f = pl.kernel(
        _sc_scatter_body,
        out_type=jax.ShapeDtypeStruct((B, 2 * C, NY, NX), jnp.float32),
        mesh=mesh,
        scratch_types=[
            pltpu.VMEM((CH * 4,), jnp.int32),
            pltpu.VMEM((SLAB,), jnp.int32),
            pltpu.VMEM((CAPT,), jnp.int32),
            pltpu.VMEM((CAPT,), jnp.int32),
            pltpu.VMEM((L, 8, NX), jnp.float32),
            pltpu.VMEM((GC, 2 * C), jnp.float32),
            pltpu.SemaphoreType.DMA,
        ],
        compiler_params=pltpu.CompilerParams(needs_layout_passes=False),
    )
    return f(vc, pf2)


BLK = 16
W_BLK = BLK * NX
HALO = 512
SEG = PLANE + 2 * HALO


def _windows(pad_ref, b, rb, xpos):
    v = pad_ref[0:1, pl.ds(b * SEG + rb * W_BLK, W_BLK + 2 * HALO)]
    rows = []
    for k in range(9):
        dy, dx = k // 3, k % 3
        delta = (dy - 1) * NX + (dx - 1)
        win = jnp.roll(v, -(HALO + delta), axis=1)[:, :W_BLK]
        if dx == 0:
            win = jnp.where(xpos == 0, 0.0, win)
        elif dx == 2:
            win = jnp.where(xpos == NX - 1, 0.0, win)
        rows.append(win)
    return rows


def _fill_pad(pad_ref, obs_ref, b, rb):
    @pl.when((b == 0) & (rb == 0))
    def _():
        for bb in range(B):
            pad_ref[0:1, pl.ds(bb * SEG, HALO)] = jnp.zeros(
                (1, HALO), jnp.float32)
            pad_ref[0:1, pl.ds(bb * SEG + HALO, PLANE)] = obs_ref[bb:bb + 1, :]
            pad_ref[0:1, pl.ds(bb * SEG + HALO + PLANE, HALO)] = jnp.zeros(
                (1, HALO), jnp.float32)


def _k1_body(obs_ref, out_ref, pad_ref):
    b = pl.program_id(0)
    rb = pl.program_id(1)
    _fill_pad(pad_ref, obs_ref, b, rb)
    xpos = lax.broadcasted_iota(jnp.int32, (1, W_BLK), 1) % NX
    rows = _windows(pad_ref, b, rb, xpos)
    rows.append(jnp.ones((1, W_BLK), jnp.float32))
    rows.append(jnp.zeros((6, W_BLK), jnp.float32))
    S = jnp.concatenate(rows, axis=0)
    contrib = lax.dot_general(S, S, (((1,), (1,)), ((), ())),
                              preferred_element_type=jnp.float32)

    @pl.when((b == 0) & (rb == 0))
    def _():
        out_ref[...] = jnp.zeros((L, L), jnp.float32)
    out_ref[...] += contrib


def _k1_moments(obs_flat):
    return pl.pallas_call(
        _k1_body,
        grid=(B, NY // BLK),
        in_specs=[pl.BlockSpec((B, PLANE), lambda b, r: (0, 0))],
        out_specs=pl.BlockSpec((L, L), lambda b, r: (0, 0)),
        out_shape=jax.ShapeDtypeStruct((L, L), jnp.float32),
        scratch_shapes=[pltpu.VMEM((1, B * SEG), jnp.float32)],
    )(obs_flat)


def _k2_body(bev_ref, obs_ref, m16_ref, w9_ref, g_ref, bt_ref, out_ref,
             pad_ref):
    del bev_ref
    b = pl.program_id(0)
    rb = pl.program_id(1)
    _fill_pad(pad_ref, obs_ref, b, rb)

    mv = m16_ref[...]
    w9 = w9_ref[...]
    cnt = mv[9:10, 9:10]
    mean = lax.dot_general(w9, mv[9:10, 0:9],
                           (((1,), (1,)), ((), ())),
                           preferred_element_type=jnp.float32) / cnt
    t1 = lax.dot_general(w9, mv[0:9, 0:9], (((1,), (0,)), ((), ())),
                         preferred_element_type=jnp.float32)
    e2 = jnp.sum(t1 * w9, axis=1, keepdims=True) / cnt
    var = e2 - mean * mean
    alpha = g_ref[...] * lax.rsqrt(var + 1e-3)
    w_eff = w9 * alpha
    b_eff = bt_ref[...] - alpha * mean

    xpos = lax.broadcasted_iota(jnp.int32, (1, W_BLK), 1) % NX
    S = jnp.concatenate(_windows(pad_ref, b, rb, xpos), axis=0)
    y = lax.dot_general(w_eff, S, (((1,), (0,)), ((), ())),
                        preferred_element_type=jnp.float32)
    y = jnp.maximum(y + b_eff, 0.0)
    out_ref[...] = y.reshape(1, C, BLK, NX)


def _k2_conv(bev_buf, obs_flat, m16, w9, g64, b64):
    return pl.pallas_call(
        _k2_body,
        grid=(B, NY // BLK),
        in_specs=[
            pl.BlockSpec(memory_space=pl.ANY),
            pl.BlockSpec((B, PLANE), lambda b, r: (0, 0)),
            pl.BlockSpec((L, L), lambda b, r: (0, 0)),
            pl.BlockSpec((C, 9), lambda b, r: (0, 0)),
            pl.BlockSpec((C, 1), lambda b, r: (0, 0)),
            pl.BlockSpec((C, 1), lambda b, r: (0, 0)),
        ],
        out_specs=pl.BlockSpec((1, C, BLK, NX), lambda b, r: (b, 1, r, 0)),
        out_shape=jax.ShapeDtypeStruct((B, 2 * C, NY, NX), jnp.float32),
        input_output_aliases={0: 0},
        scratch_shapes=[pltpu.VMEM((1, B * SEG), jnp.float32)],
    )(bev_buf, obs_flat, m16, w9, g64, b64)


def kernel(pillar_features, voxel_coords, observations, conv_w, bn_gamma,
           bn_beta):
    vc = voxel_coords.astype(jnp.int32).reshape(P * 4)
    pf2 = pillar_features.reshape(P // 2, 2 * C)
    obs_flat = observations.reshape(B, PLANE)
    w9 = conv_w.reshape(C, 9).astype(jnp.float32)
    g64 = bn_gamma.reshape(C, 1).astype(jnp.float32)
    b64 = bn_beta.reshape(C, 1).astype(jnp.float32)
    bev_buf = _sc_scatter(vc, pf2)
    m16 = _k1_moments(obs_flat)
    return _k2_conv(bev_buf, obs_flat, m16, w9, g64, b64)
